# baseline (speedup 1.0000x reference)
import functools

import jax
import jax.numpy as jnp
from jax import lax
from jax.experimental import pallas as pl
from jax.experimental.pallas import tpu as pltpu
from jax.experimental.pallas import tpu_sc as plsc

_N_NODES = 1000000
_N_FACTORS = 32
_GRP = 4
_PAD = _GRP * _N_FACTORS
_BATCH = 16384
_NC = 2
_NS = 16
_NW = _NC * _NS
_BPW = _BATCH // _NW
_CHUNK = 64
_NCHUNK = _BPW // _CHUNK

_BITREV4 = [0, 8, 4, 12, 2, 10, 6, 14, 1, 9, 5, 13, 3, 11, 7, 15]

_GDN = lax.GatherDimensionNumbers(
    offset_dims=(), collapsed_slice_dims=(0,), start_index_map=(0,))


def _perm(x, idx):
    return lax.gather(x, idx[:, None], _GDN, slice_sizes=(1,),
                      mode=lax.GatherScatterMode.PROMISE_IN_BOUNDS)


_mesh = plsc.VectorSubcoreMesh(core_axis_name="c", subcore_axis_name="s")


@functools.partial(
    pl.kernel,
    mesh=_mesh,
    out_type=jax.ShapeDtypeStruct((_BATCH,), jnp.float32),
    scratch_types=[
        pltpu.VMEM((_BPW,), jnp.int32),
        pltpu.VMEM((_BPW,), jnp.int32),
        pltpu.VMEM((_BPW,), jnp.int32),
        pltpu.VMEM((_BPW,), jnp.int32),
        pltpu.VMEM((_CHUNK, _PAD), jnp.float32),
        pltpu.VMEM((_CHUNK, _PAD), jnp.float32),
        pltpu.VMEM((_BPW,), jnp.float32),
        pltpu.SemaphoreType.DMA,
        pltpu.SemaphoreType.DMA,
    ],
)
def _dot_gather(n1_hbm, n2_hbm, table_hbm, out_hbm,
                idx1_v, idx2_v, grp1_v, grp2_v, rows1_v, rows2_v, out_v,
                sem1, sem2):
    wid = lax.axis_index("s") * _NC + lax.axis_index("c")
    base = wid * _BPW

    pltpu.sync_copy(n1_hbm.at[pl.ds(base, _BPW)], idx1_v)
    pltpu.sync_copy(n2_hbm.at[pl.ds(base, _BPW)], idx2_v)

    def grp_body(c, _):
        i0 = c * 16
        grp1_v[pl.ds(i0, 16)] = lax.shift_right_logical(
            idx1_v[pl.ds(i0, 16)], 2)
        grp2_v[pl.ds(i0, 16)] = lax.shift_right_logical(
            idx2_v[pl.ds(i0, 16)], 2)
        return 0

    lax.fori_loop(0, _BPW // 16, grp_body, 0)

    lanes = lax.iota(jnp.int32, 16)

    for ch in range(_NCHUNK):
        c0 = ch * _CHUNK
        cp1 = pltpu.async_copy(
            table_hbm.at[grp1_v.at[pl.ds(c0, _CHUNK)]], rows1_v, sem1)
        cp2 = pltpu.async_copy(
            table_hbm.at[grp2_v.at[pl.ds(c0, _CHUNK)]], rows2_v, sem2)
        cp1.wait()
        cp2.wait()

        for blk in range(_CHUNK // 16):
            i0 = blk * 16
            jv1 = jnp.bitwise_and(idx1_v[pl.ds(c0 + i0, 16)], _GRP - 1)
            jv2 = jnp.bitwise_and(idx2_v[pl.ds(c0 + i0, 16)], _GRP - 1)
            qs = []
            for ri in _BITREV4:
                j1 = lax.squeeze(lax.slice(jv1, (ri,), (ri + 1,)), (0,))
                j2 = lax.squeeze(lax.slice(jv2, (ri,), (ri + 1,)), (0,))
                o1 = j1 * _N_FACTORS
                o2 = j2 * _N_FACTORS
                k = i0 + ri
                a0 = rows1_v[k, pl.ds(o1, 16)]
                a1 = rows1_v[k, pl.ds(o1 + 16, 16)]
                b0 = rows2_v[k, pl.ds(o2, 16)]
                b1 = rows2_v[k, pl.ds(o2 + 16, 16)]
                qs.append(a0 * b0 + a1 * b1)
            vecs = qs
            for s in (8, 4, 2, 1):
                m = (lanes & s) == 0
                perm = lanes ^ s
                nxt = []
                for k2 in range(0, len(vecs), 2):
                    ta = vecs[k2] + _perm(vecs[k2], perm)
                    tb = vecs[k2 + 1] + _perm(vecs[k2 + 1], perm)
                    nxt.append(jnp.where(m, ta, tb))
                vecs = nxt
            out_v[pl.ds(c0 + i0, 16)] = vecs[0]

    pltpu.sync_copy(out_v, out_hbm.at[pl.ds(base, _BPW)])


def kernel(node1, node2, node_factors):
    grouped = node_factors.reshape(_N_NODES // _GRP, _PAD)
    return _dot_gather(node1, node2, grouped)


# speedup vs baseline: 1.6793x; 1.6793x over previous
import functools

import jax
import jax.numpy as jnp
from jax import lax
from jax.experimental import pallas as pl
from jax.experimental.pallas import tpu as pltpu
from jax.experimental.pallas import tpu_sc as plsc

_N_FACTORS = 32
_BATCH = 16384
_NC = 2
_NS = 16
_NW = _NC * _NS
_BPW = _BATCH // _NW
_CHUNK = _BPW // 2
_NSEM = 8

_BITREV4 = [0, 8, 4, 12, 2, 10, 6, 14, 1, 9, 5, 13, 3, 11, 7, 15]

_GDN = lax.GatherDimensionNumbers(
    offset_dims=(), collapsed_slice_dims=(0,), start_index_map=(0,))


def _perm(x, idx):
    return lax.gather(x, idx[:, None], _GDN, slice_sizes=(1,),
                      mode=lax.GatherScatterMode.PROMISE_IN_BOUNDS)


_mesh = plsc.VectorSubcoreMesh(core_axis_name="c", subcore_axis_name="s")


@functools.partial(
    pl.kernel,
    mesh=_mesh,
    out_type=jax.ShapeDtypeStruct((_BATCH,), jnp.float32),
    scratch_types=[
        pltpu.VMEM((_BPW,), jnp.int32),
        pltpu.VMEM((_BPW,), jnp.int32),
        pltpu.VMEM((_CHUNK, _N_FACTORS), jnp.float32),
        pltpu.VMEM((_CHUNK, _N_FACTORS), jnp.float32),
        pltpu.VMEM((_BPW,), jnp.float32),
        [pltpu.SemaphoreType.DMA] * _NSEM,
    ],
)
def _dot_gather(n1_hbm, n2_hbm, table_hbm, out_hbm,
                idx1_v, idx2_v, rows1_v, rows2_v, out_v, sems):
    wid = lax.axis_index("s") * _NC + lax.axis_index("c")
    base = wid * _BPW

    pltpu.sync_copy(n1_hbm.at[pl.ds(base, _BPW)], idx1_v)
    pltpu.sync_copy(n2_hbm.at[pl.ds(base, _BPW)], idx2_v)

    lanes = lax.iota(jnp.int32, 16)

    for half in range(2):
        h0 = half * _CHUNK

        def gather_body(c, _):
            i0 = c * 16
            vec1 = idx1_v[pl.ds(h0 + i0, 16)]
            vec2 = idx2_v[pl.ds(h0 + i0, 16)]
            for k in range(16):
                r1 = lax.squeeze(lax.slice(vec1, (k,), (k + 1,)), (0,))
                r2 = lax.squeeze(lax.slice(vec2, (k,), (k + 1,)), (0,))
                pltpu.async_copy(table_hbm.at[pl.ds(r1, 1), :],
                                 rows1_v.at[pl.ds(i0 + k, 1), :],
                                 sems[k % _NSEM])
                pltpu.async_copy(table_hbm.at[pl.ds(r2, 1), :],
                                 rows2_v.at[pl.ds(i0 + k, 1), :],
                                 sems[(k + 1) % _NSEM])
            return 0

        lax.fori_loop(0, _CHUNK // 16, gather_body, 0)

        per_sem = 2 * _CHUNK // _NSEM
        for k in range(_NSEM):
            pltpu.make_async_copy(
                table_hbm.at[pl.ds(0, per_sem), :],
                rows1_v.at[pl.ds(0, per_sem), :], sems[k]).wait()

        def blk_body(blk, _):
            i0 = blk * 16
            qs = []
            for ri in _BITREV4:
                r = i0 + ri
                a0 = rows1_v[r, pl.ds(0, 16)]
                a1 = rows1_v[r, pl.ds(16, 16)]
                b0 = rows2_v[r, pl.ds(0, 16)]
                b1 = rows2_v[r, pl.ds(16, 16)]
                qs.append(a0 * b0 + a1 * b1)
            vecs = qs
            for s in (8, 4, 2, 1):
                m = (lanes & s) == 0
                perm = lanes ^ s
                nxt = []
                for k in range(0, len(vecs), 2):
                    ta = vecs[k] + _perm(vecs[k], perm)
                    tb = vecs[k + 1] + _perm(vecs[k + 1], perm)
                    nxt.append(jnp.where(m, ta, tb))
                vecs = nxt
            out_v[pl.ds(h0 + i0, 16)] = vecs[0]
            return 0

        lax.fori_loop(0, _CHUNK // 16, blk_body, 0)

    pltpu.sync_copy(out_v, out_hbm.at[pl.ds(base, _BPW)])


def kernel(node1, node2, node_factors):
    return _dot_gather(node1, node2, node_factors)
